# baseline (speedup 1.0000x reference)
import functools

import jax
import jax.numpy as jnp
from jax import lax
from jax.experimental import pallas as pl
from jax.experimental.pallas import tpu as pltpu
from jax.experimental.pallas import tpu_sc as plsc

D = 64
T = 200
B = 4096
NW = 32
NCHUNK = B // NW
NBUF = 4
NIDX = 2

_mesh = plsc.VectorSubcoreMesh(core_axis_name="c", subcore_axis_name="s")


@functools.partial(
    pl.kernel,
    out_type=jax.ShapeDtypeStruct((B * T, D), jnp.float32),
    mesh=_mesh,
    scratch_types=[
        pltpu.VMEM((NCHUNK * NIDX, 100), jnp.int32),
        [pltpu.VMEM((T, D), jnp.float32) for _ in range(NBUF)],
        pltpu.VMEM((T, D), jnp.float32),
        [pltpu.SemaphoreType.DMA for _ in range(NBUF)],
        [pltpu.SemaphoreType.DMA for _ in range(NBUF)],
    ],
    compiler_params=pltpu.CompilerParams(use_tc_tiling_on_sc=False),
)
def _emb(idx_hbm, word_hbm, pos_hbm, out_hbm, idx_v, rows, pos_v, gsem, osem):
    wid = lax.axis_index("s") * 2 + lax.axis_index("c")
    pltpu.sync_copy(pos_hbm, pos_v)
    pltpu.sync_copy(idx_hbm.at[pl.ds(wid * NCHUNK * NIDX, NCHUNK * NIDX)], idx_v)

    def fire_gather(c, b):
        for i in range(NIDX):
            pltpu.async_copy(
                word_hbm.at[idx_v.at[c * NIDX + i]],
                rows[b].at[pl.ds(i * 100, 100)],
                gsem[b],
            )

    def drain(sem, b):
        pltpu.make_async_copy(word_hbm.at[pl.ds(0, T)], rows[b], sem).wait()

    fire_gather(0, 0)
    fire_gather(1, 1)

    def ring_body(jj, carry):
        for k in range(NBUF):
            c = jj * NBUF + k
            drain(gsem[k], k)

            def add_body(r, c2):
                for col in range(D // 16):
                    sl = pl.ds(col * 16, 16)
                    rows[k][r, sl] = rows[k][r, sl] + pos_v[r, sl]
                return c2

            lax.fori_loop(0, T, add_body, 0, unroll=4)
            pltpu.async_copy(
                rows[k],
                out_hbm.at[pl.ds((wid * NCHUNK + c) * T, T)],
                osem[k],
            )
            b2 = (k + 2) % NBUF
            if k < 2:
                @pl.when(jj > 0)
                def _():
                    drain(osem[b2], b2)

                fire_gather(c + 2, b2)
            else:
                @pl.when(jj < NCHUNK // NBUF - 1)
                def _():
                    drain(osem[b2], b2)
                    fire_gather(c + 2, b2)
        return carry

    lax.fori_loop(0, NCHUNK // NBUF, ring_body, 0)
    for b in range(NBUF):
        drain(osem[b], b)


def kernel(sentence, word_emb, pos_emb):
    idx = jnp.transpose(sentence, (1, 0)).reshape(B * T // 100, 100)
    pos_slice = lax.slice(pos_emb, (1, 0), (T + 1, D))
    out = _emb(idx, word_emb, pos_slice)
    return out.reshape(B, T, D)
